# baseline (speedup 1.0000x reference)
import functools

import jax
import jax.numpy as jnp
from jax import lax
from jax.experimental import pallas as pl
from jax.experimental.pallas import tpu as pltpu
from jax.experimental.pallas import tpu_sc as plsc

CHUNK = 128


def _gather_flat(table_flat, ids, V, D):
    B = ids.shape[0]
    info = plsc.get_sparse_core_info()
    NC, NS = info.num_cores, info.num_subcores
    L = info.num_lanes
    NW = NC * NS
    b_per_w = B // NW
    n_idx = b_per_w * D
    mesh = plsc.VectorSubcoreMesh(core_axis_name="c", subcore_axis_name="s")
    ids2 = ids.astype(jnp.int32).reshape(NW, b_per_w)

    @functools.partial(
        pl.kernel,
        mesh=mesh,
        compiler_params=pltpu.CompilerParams(use_tc_tiling_on_sc=False,
                                             needs_layout_passes=False),
        out_type=jax.ShapeDtypeStruct((B * D,), jnp.float32),
        scratch_types=[
            pltpu.VMEM((b_per_w,), jnp.int32),
            pltpu.VMEM((n_idx,), jnp.int32),
            pltpu.VMEM((n_idx,), jnp.float32),
            pltpu.SemaphoreType.DMA,
        ],
    )
    def _emb(tab, ids_hbm, out, vidx, idxbuf, rows, sem):
        wid = lax.axis_index("s") * NC + lax.axis_index("c")
        pltpu.sync_copy(ids_hbm.at[wid], vidx)

        lane_offs = lax.iota(jnp.int32, L) * D

        @pl.loop(0, b_per_w // L)
        def _build(k):
            v = vidx[pl.ds(k * L, L)]
            pos0 = lane_offs + k * (L * D)
            for c in range(D):
                plsc.store_scatter(idxbuf, [pos0 + c], v + c * V)

        copies = []
        for q in range(n_idx // CHUNK):
            copies.append(pltpu.async_copy(
                tab.at[idxbuf.at[pl.ds(q * CHUNK, CHUNK)]],
                rows.at[pl.ds(q * CHUNK, CHUNK)], sem))
        for c in copies:
            c.wait()

        pltpu.sync_copy(rows, out.at[pl.ds(wid * n_idx, n_idx)])

    return _emb(table_flat, ids2).reshape(B, D)


def kernel(chem_id, dis_id, chem_emb, disease_emb):
    Vc, D = chem_emb.shape
    Vd = disease_emb.shape[0]
    chem_x = _gather_flat(chem_emb.T.reshape(-1), chem_id, Vc, D)
    dis_x = _gather_flat(disease_emb.T.reshape(-1), dis_id, Vd, D)
    return (chem_x, dis_x)
